# Initial kernel scaffold; baseline (speedup 1.0000x reference)
import jax
import jax.numpy as jnp
from jax.experimental import pallas as pl


def kernel(x_paper, x_author, ei_cites, ei_writes, ei_rev, Wl0_cites, bl0_cites, Wr0_cites, Wl0_writes, bl0_writes, Wr0_writes, Wl0_rev, bl0_rev, Wr0_rev, Wl1_cites, bl1_cites, Wr1_cites, Wl1_writes, bl1_writes, Wr1_writes, Wl1_rev, bl1_rev, Wr1_rev, W_out, b_out):
    raise NotImplementedError("write your pallas kernel here")


# speedup vs baseline: 6.2314x; 6.2314x over previous
import functools

import jax
import jax.numpy as jnp
from jax import lax
from jax.experimental import pallas as pl
from jax.experimental.pallas import tpu as pltpu
from jax.experimental.pallas import tpu_sc as plsc

N = 10000
NPAD = 10240
D = 128
DH = 64
E = 640000
OUT = 64
NC = 2
NS = 16
NW = NC * NS
EPW = E // NW
CH = 80
NCH = EPW // CH
RPT = NPAD // NS
CW = 8
BM = 640


def _sc_agg(ntypes, with_counts):
  tab_map = (0, 1, 0)[:ntypes]
  mesh = plsc.VectorSubcoreMesh(core_axis_name="c", subcore_axis_name="s")
  out_type = [jax.ShapeDtypeStruct((NC, NPAD, DH), jnp.float32)
              for _ in range(2 * ntypes)]
  if with_counts:
    out_type += [jax.ShapeDtypeStruct((NC, NPAD, CW), jnp.float32)
                 for _ in range(ntypes)]
  scratch_types = [
      pltpu.VMEM((NCH, CH), jnp.int32),
      pltpu.VMEM((NCH, CH), jnp.int32),
      pltpu.VMEM((CH, DH), jnp.float32),
      pltpu.VMEM((CH, CW), jnp.float32),
      pltpu.VMEM_SHARED((NPAD, DH), jnp.float32),
      pltpu.VMEM_SHARED((NPAD, CW), jnp.float32),
      pltpu.SemaphoreType.DMA,
  ]

  def body(*refs):
    nt = ntypes
    srcs = refs[0:2 * nt:2]
    dsts = refs[1:2 * nt:2]
    tabs = refs[2 * nt:2 * nt + 4]
    ones_h, zf_h, zc_h = refs[2 * nt + 4:2 * nt + 7]
    o = 2 * nt + 7
    aggs = refs[o:o + 2 * nt]
    o += 2 * nt
    cnts = refs[o:o + nt] if with_counts else ()
    o += nt if with_counts else 0
    src_v, dst_v, rows_v, ones_v, acc, cacc, sem = refs[o:]

    c = lax.axis_index("c")
    s = lax.axis_index("s")
    wid = c * NS + s
    if with_counts:
      pltpu.sync_copy(ones_h, ones_v)

    for t in range(nt):
      pltpu.sync_copy(srcs[t].at[wid], src_v)
      pltpu.sync_copy(dsts[t].at[wid], dst_v)
      for h in range(2):
        tab = tabs[2 * tab_map[t] + h]
        do_cnt = with_counts and h == 0
        pltpu.sync_copy(zf_h, acc.at[pl.ds(s * RPT, RPT), :])
        if do_cnt:
          pltpu.sync_copy(zc_h, cacc.at[pl.ds(s * RPT, RPT), :])
        plsc.subcore_barrier()

        def chunk(i, carry):
          pltpu.async_copy(tab.at[src_v.at[i]], rows_v, sem).wait()
          pltpu.sync_copy(rows_v, acc.at[dst_v.at[i]], add=True)
          if do_cnt:
            pltpu.sync_copy(ones_v, cacc.at[dst_v.at[i]], add=True)
          return carry

        lax.fori_loop(0, NCH, chunk, 0)
        plsc.subcore_barrier()
        pltpu.sync_copy(acc.at[pl.ds(s * RPT, RPT), :],
                        aggs[2 * t + h].at[c, pl.ds(s * RPT, RPT), :])
        if do_cnt:
          pltpu.sync_copy(cacc.at[pl.ds(s * RPT, RPT), :],
                          cnts[t].at[c, pl.ds(s * RPT, RPT), :])
        plsc.subcore_barrier()

  return pl.kernel(body, out_type=out_type, mesh=mesh,
                   scratch_types=scratch_types,
                   compiler_params=pltpu.CompilerParams(
                       use_tc_tiling_on_sc=False))


_sc_layer0 = _sc_agg(3, True)
_sc_layer1 = _sc_agg(2, False)


def _mean(a0, a1, cnt):
  cc = jnp.maximum(cnt[0, :, 0:1] + cnt[1, :, 0:1], 1.0)
  return jnp.concatenate([a0[0] + a0[1], a1[0] + a1[1]], axis=1) / cc


def _dot(a, b):
  return jnp.dot(a, b, preferred_element_type=jnp.float32)


def _tc0_body(aC0, aC1, cntC, aW0, aW1, cntW, aR0, aR1, cntR, xp, xa,
              wlc, wlw, wlr, wrc, wrw, wrr, bc, bw, br,
              xp_o0, xp_o1, xa_o0, xa_o1):
  p = (_dot(_mean(aC0, aC1, cntC), wlc[...])
       + _dot(_mean(aW0, aW1, cntW), wlw[...])
       + _dot(xp[...], wrc[...] + wrw[...]) + bc[...] + bw[...])
  p = jnp.maximum(p, 0.0)
  xp_o0[...] = p[:, :DH]
  xp_o1[...] = p[:, DH:]
  a = (_dot(_mean(aR0, aR1, cntR), wlr[...])
       + _dot(xa[...], wrr[...]) + br[...])
  a = jnp.maximum(a, 0.0)
  xa_o0[...] = a[:, :DH]
  xa_o1[...] = a[:, DH:]


def _tc1_body(aC0, aC1, cntC, aW0, aW1, cntW, xp0, xp1,
              wlc, wlw, wrc, wrw, bc, bw, wout, bout, out_o):
  xp = jnp.concatenate([xp0[...], xp1[...]], axis=1)
  p = (_dot(_mean(aC0, aC1, cntC), wlc[...])
       + _dot(_mean(aW0, aW1, cntW), wlw[...])
       + _dot(xp, wrc[...] + wrw[...]) + bc[...] + bw[...])
  out_o[...] = _dot(jnp.maximum(p, 0.0), wout[...]) + bout[...]


_AGG_BS = pl.BlockSpec((NC, BM, DH), lambda i: (0, i, 0))
_CNT_BS = pl.BlockSpec((NC, BM, CW), lambda i: (0, i, 0))
_X_BS = pl.BlockSpec((BM, D), lambda i: (i, 0))
_XH_BS = pl.BlockSpec((BM, DH), lambda i: (i, 0))
_W_BS = pl.BlockSpec((D, D), lambda i: (0, 0))
_B_BS = pl.BlockSpec((1, D), lambda i: (0, 0))

_tc0 = pl.pallas_call(
    _tc0_body,
    grid=(NPAD // BM,),
    in_specs=[_AGG_BS, _AGG_BS, _CNT_BS, _AGG_BS, _AGG_BS, _CNT_BS,
              _AGG_BS, _AGG_BS, _CNT_BS, _X_BS, _X_BS,
              _W_BS, _W_BS, _W_BS, _W_BS, _W_BS, _W_BS,
              _B_BS, _B_BS, _B_BS],
    out_specs=[_XH_BS, _XH_BS, _XH_BS, _XH_BS],
    out_shape=[jax.ShapeDtypeStruct((NPAD, DH), jnp.float32)
               for _ in range(4)],
)

_tc1 = pl.pallas_call(
    _tc1_body,
    grid=(NPAD // BM,),
    in_specs=[_AGG_BS, _AGG_BS, _CNT_BS, _AGG_BS, _AGG_BS, _CNT_BS,
              _XH_BS, _XH_BS,
              _W_BS, _W_BS, _W_BS, _W_BS, _B_BS, _B_BS,
              pl.BlockSpec((D, OUT), lambda i: (0, 0)),
              pl.BlockSpec((1, OUT), lambda i: (0, 0))],
    out_specs=pl.BlockSpec((BM, OUT), lambda i: (i, 0)),
    out_shape=jax.ShapeDtypeStruct((NPAD, OUT), jnp.float32),
)


def kernel(x_paper, x_author, ei_cites, ei_writes, ei_rev,
           Wl0_cites, bl0_cites, Wr0_cites, Wl0_writes, bl0_writes,
           Wr0_writes, Wl0_rev, bl0_rev, Wr0_rev,
           Wl1_cites, bl1_cites, Wr1_cites, Wl1_writes, bl1_writes,
           Wr1_writes, Wl1_rev, bl1_rev, Wr1_rev, W_out, b_out):
  f32 = jnp.float32
  xp = jnp.pad(x_paper, ((0, NPAD - N), (0, 0)))
  xa = jnp.pad(x_author, ((0, NPAD - N), (0, 0)))
  rs = lambda v: v.reshape(NW, NCH, CH)
  sC, dC = rs(ei_cites[0]), rs(ei_cites[1])
  sW, dW = rs(ei_writes[0]), rs(ei_writes[1])
  sR, dR = rs(ei_rev[0]), rs(ei_rev[1])
  ones = jnp.ones((CH, CW), f32)
  zf = jnp.zeros((RPT, DH), f32)
  zc = jnp.zeros((RPT, CW), f32)
  r1 = lambda b: b.reshape(1, -1)

  (aC0, aC1, aW0, aW1, aR0, aR1, cntC, cntW, cntR) = _sc_layer0(
      sC, dC, sW, dW, sR, dR,
      xp[:, :DH], xp[:, DH:], xa[:, :DH], xa[:, DH:], ones, zf, zc)
  xp0, xp1, xa0, xa1 = _tc0(
      aC0, aC1, cntC, aW0, aW1, cntW, aR0, aR1, cntR, xp, xa,
      Wl0_cites, Wl0_writes, Wl0_rev, Wr0_cites, Wr0_writes, Wr0_rev,
      r1(bl0_cites), r1(bl0_writes), r1(bl0_rev))
  (bC0, bC1, bW0, bW1) = _sc_layer1(
      sC, dC, sW, dW, xp0, xp1, xa0, xa1, ones, zf, zc)
  out = _tc1(bC0, bC1, cntC, bW0, bW1, cntW, xp0, xp1,
             Wl1_cites, Wl1_writes, Wr1_cites, Wr1_writes,
             r1(bl1_cites), r1(bl1_writes), W_out, r1(b_out))
  return out[:N]
